# baseline (speedup 1.0000x reference)
import functools

import jax
import jax.numpy as jnp
import numpy as np
from jax import lax
from jax.experimental import pallas as pl
from jax.experimental.pallas import tpu as pltpu
from jax.experimental.pallas import tpu_sc as plsc

_N = 100000
_C = 128
_MAX_BINS = 32
_ROWS = 1000
_GRID = _N // _ROWS

_NW = 32
_CHUNK = 3136
_VECS = _CHUNK // 16
_PAD_N = _NW * _CHUNK
_SENT = np.int32(2**31 - 1)

_LP = 512
_HB = 65536


def _bin_edges(n, n_bins):
    spb = n // n_bins
    rem = n % n_bins
    edges = []
    start = 0
    for i in range(n_bins):
        size = spb + (1 if i < rem else 0)
        end = start + size
        if start >= n:
            break
        edges.append((start, end))
        start = end
    return edges


_RANKS = np.array(
    sorted({s for b in range(1, _MAX_BINS + 1)
            for (s, e) in _bin_edges(_N, b)} - {0}), dtype=np.int32)
_KP = len(_RANKS)
_KROWS = 400

_RANK_FULL = np.concatenate([[0], _RANKS, [_N]])
_RANK_POS = {int(r): i for i, r in enumerate(_RANK_FULL)}

_PAIR_B, _PAIR_S, _PAIR_E = [], [], []
for _b in range(1, _MAX_BINS + 1):
    for _s, _e in _bin_edges(_N, _b):
        _PAIR_B.append(_b)
        _PAIR_S.append(_RANK_POS[_s])
        _PAIR_E.append(_RANK_POS[_e])
_PAIR_B = np.array(_PAIR_B)
_PAIR_S = np.array(_PAIR_S)
_PAIR_E = np.array(_PAIR_E)
_PAIR_LEN = (_RANK_FULL[_PAIR_E] - _RANK_FULL[_PAIR_S]).astype(np.float32)
_ADJ = np.array([i for i in range(1, len(_PAIR_B))
                 if _PAIR_B[i] == _PAIR_B[i - 1]])
_VIOL_MAT = np.zeros((_MAX_BINS + 1, len(_ADJ)), np.float32)
for _i, _a in enumerate(_ADJ):
    _VIOL_MAT[_PAIR_B[_a], _i] = 1.0
_ECE_MAT = np.zeros((_MAX_BINS + 1, len(_PAIR_B)), np.float32)
for _i, _b in enumerate(_PAIR_B):
    _ECE_MAT[_b, _i] = 1.0


def _stage_a_body(logits_ref, labels_ref, conf_ref, corr_ref):
    x = logits_ref[0]
    m = jnp.max(x, axis=1, keepdims=True)
    q = jnp.exp(x - m)
    s = jnp.sum(q, axis=1)
    conf = 1.0 / s
    lab = labels_ref[0, 0]
    lane = lax.broadcasted_iota(jnp.int32, x.shape, 1)
    xl = jnp.max(jnp.where(lane == lab[:, None], x, -jnp.inf), axis=1)
    corr = (xl == m[:, 0]).astype(jnp.float32)
    conf_ref[0, 0] = conf
    corr_ref[0, 0] = corr


def _stage_a(logits, labels):
    logits3 = logits.reshape(_GRID, _ROWS, _C)
    labels3 = labels.reshape(_GRID, 1, _ROWS)
    conf, corr = pl.pallas_call(
        _stage_a_body,
        grid=(_GRID,),
        in_specs=[
            pl.BlockSpec((1, _ROWS, _C), lambda i: (i, 0, 0)),
            pl.BlockSpec((1, 1, _ROWS), lambda i: (i, 0, 0)),
        ],
        out_specs=[
            pl.BlockSpec((1, 1, _ROWS), lambda i: (i, 0, 0)),
            pl.BlockSpec((1, 1, _ROWS), lambda i: (i, 0, 0)),
        ],
        out_shape=[
            jax.ShapeDtypeStruct((_GRID, 1, _ROWS), jnp.float32),
            jax.ShapeDtypeStruct((_GRID, 1, _ROWS), jnp.float32),
        ],
    )(logits3, labels3)
    return conf.reshape(_N), corr.reshape(_N)


_MESH = plsc.VectorSubcoreMesh(core_axis_name="c", subcore_axis_name="s")


def _wid():
    return lax.axis_index("s") * 2 + lax.axis_index("c")


def _search_le(list_ref, v):
    lo = jnp.zeros((16,), jnp.int32)
    step = _LP // 2
    while step >= 1:
        probe = lo + (step - 1)
        t = plsc.load_gather(list_ref, [probe])
        lo = lo + jnp.where(t <= v, step, 0)
        step //= 2
    return lo


def _search_le4(list_ref, vs):
    los = [jnp.zeros((16,), jnp.int32) for _ in vs]
    step = _LP // 2
    while step >= 1:
        ts = [plsc.load_gather(list_ref, [lo + (step - 1)]) for lo in los]
        los = [lo + jnp.where(t <= v, step, 0)
               for lo, t, v in zip(los, ts, vs)]
        step //= 2
    return los


def _zero_ref(ref, nwords):
    z = jnp.zeros((16,), ref.dtype)

    def body(i, _):
        ref[pl.ds(i * 16, 16)] = z
        return 0

    lax.fori_loop(0, nwords // 16, body, 0)


@functools.partial(
    pl.kernel,
    out_type=jax.ShapeDtypeStruct((_NW, _HB), jnp.int32),
    mesh=_MESH,
    compiler_params=pltpu.CompilerParams(needs_layout_passes=False),
    scratch_types=[
        pltpu.VMEM((_CHUNK,), jnp.int32),
        pltpu.VMEM((_HB,), jnp.int32),
    ],
)
def _pass_a(key_hbm, out_hbm, keys_v, hist_v):
    w = _wid()
    pltpu.sync_copy(key_hbm.at[pl.ds(w * _CHUNK, _CHUNK)], keys_v)
    _zero_ref(hist_v, _HB)
    one = jnp.ones((16,), jnp.int32)
    full = jnp.ones((16,), jnp.bool_)

    def body(i, _):
        for j in range(4):
            k = keys_v[pl.ds((i * 4 + j) * 16, 16)]
            b = lax.shift_right_logical(k, 16)
            plsc.addupdate_scatter(hist_v, [b], one, mask=full)
        return 0

    lax.fori_loop(0, _VECS // 4, body, 0)
    pltpu.sync_copy(hist_v, out_hbm.at[w])


def _make_refine(shift):
    @functools.partial(
        pl.kernel,
        out_type=jax.ShapeDtypeStruct((_NW, _KROWS * 256), jnp.int32),
        mesh=_MESH,
        compiler_params=pltpu.CompilerParams(needs_layout_passes=False),
        scratch_types=[
            pltpu.VMEM((_CHUNK,), jnp.int32),
            pltpu.VMEM((_LP,), jnp.int32),
            pltpu.VMEM((_KROWS * 256,), jnp.int32),
        ],
    )
    def refine(key_hbm, list_hbm, out_hbm, keys_v, list_v, hist_v):
        w = _wid()
        pltpu.sync_copy(key_hbm.at[pl.ds(w * _CHUNK, _CHUNK)], keys_v)
        pltpu.sync_copy(list_hbm, list_v)
        _zero_ref(hist_v, _KROWS * 256)
        one = jnp.ones((16,), jnp.int32)

        def body(i, _):
            ks = [keys_v[pl.ds((i * 4 + j) * 16, 16)] for j in range(4)]
            pfxs = [lax.shift_right_logical(k, 8 + shift) for k in ks]
            ss = _search_le4(list_v, pfxs)
            for k, pfx, s in zip(ks, pfxs, ss):
                slot = jnp.maximum(s - 1, 0)
                pv = plsc.load_gather(list_v, [slot])
                valid = jnp.logical_and(s >= 1, pv == pfx)
                dig = jnp.bitwise_and(lax.shift_right_logical(k, shift), 255)
                flat = jnp.where(valid, slot * 256 + dig, 0)
                plsc.addupdate_scatter(hist_v, [flat], one, mask=valid)
            return 0

        lax.fori_loop(0, _VECS // 4, body, 0)
        pltpu.sync_copy(hist_v, out_hbm.at[w])

    return refine


_pass_b = _make_refine(8)
_pass_c = _make_refine(0)


@functools.partial(
    pl.kernel,
    out_type=jax.ShapeDtypeStruct((_NW, 5 * _LP), jnp.float32),
    mesh=_MESH,
    compiler_params=pltpu.CompilerParams(needs_layout_passes=False),
    scratch_types=[
        pltpu.VMEM((_CHUNK,), jnp.int32),
        pltpu.VMEM((_CHUNK,), jnp.float32),
        pltpu.VMEM((_CHUNK,), jnp.float32),
        pltpu.VMEM((_LP,), jnp.int32),
        pltpu.VMEM((5 * _LP,), jnp.float32),
    ],
)
def _pass_d(key_hbm, conf_hbm, corr_hbm, list_hbm, out_hbm,
            keys_v, conf_v, corr_v, list_v, seg_v):
    w = _wid()
    base = w * _CHUNK
    pltpu.sync_copy(key_hbm.at[pl.ds(base, _CHUNK)], keys_v)
    pltpu.sync_copy(conf_hbm.at[pl.ds(base, _CHUNK)], conf_v)
    pltpu.sync_copy(corr_hbm.at[pl.ds(base, _CHUNK)], corr_v)
    pltpu.sync_copy(list_hbm, list_v)
    _zero_ref(seg_v, 5 * _LP)
    onef = jnp.ones((16,), jnp.float32)
    lanes = lax.iota(jnp.int32, 16)

    def body(i, _):
        ks = [keys_v[pl.ds((i * 4 + j) * 16, 16)] for j in range(4)]
        ss = _search_le4(list_v, ks)
        for j, (k, s) in enumerate(zip(ks, ss)):
            cf = conf_v[pl.ds((i * 4 + j) * 16, 16)]
            cr = corr_v[pl.ds((i * 4 + j) * 16, 16)]
            idx = base + (i * 4 + j) * 16 + lanes
            inb = idx < _N
            s = jnp.minimum(s, _KP)
            plsc.addupdate_scatter(seg_v, [s], onef, mask=inb)
            plsc.addupdate_scatter(seg_v, [_LP + s], cf, mask=inb)
            plsc.addupdate_scatter(seg_v, [2 * _LP + s], cr, mask=inb)
            slot = jnp.maximum(s - 1, 0)
            tv = plsc.load_gather(list_v, [slot])
            tie = jnp.logical_and(jnp.logical_and(s >= 1, tv == k), inb)
            plsc.addupdate_scatter(seg_v, [3 * _LP + slot], onef, mask=tie)
            plsc.addupdate_scatter(seg_v, [4 * _LP + slot], cr, mask=tie)
        return 0

    lax.fori_loop(0, _VECS // 4, body, 0)
    pltpu.sync_copy(seg_v, out_hbm.at[w])


def kernel(logits, labels):
    conf, corr = _stage_a(logits, labels)
    key = lax.bitcast_convert_type(conf, jnp.int32)

    padlen = _PAD_N - _N
    key_p = jnp.concatenate([key, jnp.full((padlen,), _SENT, jnp.int32)])
    conf_p = jnp.concatenate([conf, jnp.zeros((padlen,), jnp.float32)])
    corr_p = jnp.concatenate([corr, jnp.zeros((padlen,), jnp.float32)])

    ranks = jnp.asarray(_RANKS)

    hist_a = _pass_a(key_p).sum(axis=0)
    cum_a = jnp.cumsum(hist_a)
    p16 = jnp.searchsorted(cum_a, ranks, side="right").astype(jnp.int32)
    base16 = jnp.where(p16 > 0, cum_a[jnp.maximum(p16 - 1, 0)], 0)

    def padlist(v):
        return jnp.concatenate(
            [v, jnp.full((_LP - _KP,), _SENT, jnp.int32)])

    hist_b = _pass_b(key_p, padlist(p16)).sum(axis=0).reshape(_KROWS, 256)
    cum_b = jnp.cumsum(hist_b[:_KP], axis=1)
    rs_b = jnp.searchsorted(p16, p16, side="right").astype(jnp.int32) - 1
    row_b = cum_b[rs_b]
    m_b = (ranks - base16)[:, None]
    d_b = (row_b <= m_b).sum(axis=1).astype(jnp.int32)
    base24 = base16 + jnp.where(
        d_b > 0, row_b[jnp.arange(_KP), jnp.maximum(d_b - 1, 0)], 0)
    p24 = p16 * 256 + d_b

    hist_c = _pass_c(key_p, padlist(p24)).sum(axis=0).reshape(_KROWS, 256)
    cum_c = jnp.cumsum(hist_c[:_KP], axis=1)
    rs_c = jnp.searchsorted(p24, p24, side="right").astype(jnp.int32) - 1
    row_c = cum_c[rs_c]
    m_c = (ranks - base24)[:, None]
    d_c = (row_c <= m_c).sum(axis=1).astype(jnp.int32)
    tkeys = p24 * 256 + d_c

    segs = _pass_d(key_p, conf_p, corr_p,
                   padlist(tkeys)).sum(axis=0).reshape(5, _LP)
    seg_cnt, seg_conf, seg_corr = segs[0], segs[1], segs[2]
    eq_cnt, eq_corr = segs[3], segs[4]

    cum_cnt = jnp.cumsum(seg_cnt)
    cum_conf = jnp.cumsum(seg_conf)
    cum_corr = jnp.cumsum(seg_corr)

    rs_t = jnp.searchsorted(tkeys, tkeys, side="right").astype(jnp.int32) - 1
    take = ranks.astype(jnp.float32) - cum_cnt[:_KP]
    conf_val = lax.bitcast_convert_type(tkeys, jnp.float32)
    ecnt = eq_cnt[rs_t]
    tie_avg = jnp.where(ecnt > 0, eq_corr[rs_t] / jnp.maximum(ecnt, 1.0), 0.0)
    pc_in = cum_conf[:_KP] + take * conf_val
    pr_in = cum_corr[:_KP] + take * tie_avg

    total_conf = cum_conf[_KP]
    total_corr = cum_corr[_KP]
    pc = jnp.concatenate([jnp.zeros((1,)), pc_in, total_conf[None]])
    pr = jnp.concatenate([jnp.zeros((1,)), pr_in, total_corr[None]])

    pair_s = jnp.asarray(_PAIR_S)
    pair_e = jnp.asarray(_PAIR_E)
    ln = jnp.asarray(_PAIR_LEN)
    avg_a = (pr[pair_e] - pr[pair_s]) / ln
    avg_c = (pc[pair_e] - pc[pair_s]) / ln

    adj = jnp.asarray(_ADJ)
    viol = (avg_a[adj] < avg_a[adj - 1]).astype(jnp.float32)
    viol_b = jnp.asarray(_VIOL_MAT) @ viol
    b_arr = jnp.arange(_MAX_BINS + 1)
    n_bins = jnp.min(jnp.where(viol_b > 0, b_arr - 1, _MAX_BINS))

    terms = (ln / _N) * jnp.abs(avg_c - avg_a)
    ece_b = jnp.asarray(_ECE_MAT) @ terms
    ece_b = ece_b.at[1].set(0.0)
    return ece_b[n_bins].astype(jnp.float32)


# speedup vs baseline: 1.0412x; 1.0412x over previous
import functools

import jax
import jax.numpy as jnp
import numpy as np
from jax import lax
from jax.experimental import pallas as pl
from jax.experimental.pallas import tpu as pltpu
from jax.experimental.pallas import tpu_sc as plsc

_N = 100000
_C = 128
_MAX_BINS = 32
_ROWS = 1000
_GRID = _N // _ROWS

_NW = 32
_CHUNK = 3136
_VECS = _CHUNK // 16
_PAD_N = _NW * _CHUNK
_SENT = np.int32(2**31 - 1)

_LP = 512
_HB = 65536


def _bin_edges(n, n_bins):
    spb = n // n_bins
    rem = n % n_bins
    edges = []
    start = 0
    for i in range(n_bins):
        size = spb + (1 if i < rem else 0)
        end = start + size
        if start >= n:
            break
        edges.append((start, end))
        start = end
    return edges


_RANKS = np.array(
    sorted({s for b in range(1, _MAX_BINS + 1)
            for (s, e) in _bin_edges(_N, b)} - {0}), dtype=np.int32)
_KP = len(_RANKS)
_KROWS = 400

_RANK_FULL = np.concatenate([[0], _RANKS, [_N]])
_RANK_POS = {int(r): i for i, r in enumerate(_RANK_FULL)}

_PAIR_B, _PAIR_S, _PAIR_E = [], [], []
for _b in range(1, _MAX_BINS + 1):
    for _s, _e in _bin_edges(_N, _b):
        _PAIR_B.append(_b)
        _PAIR_S.append(_RANK_POS[_s])
        _PAIR_E.append(_RANK_POS[_e])
_PAIR_B = np.array(_PAIR_B)
_PAIR_S = np.array(_PAIR_S)
_PAIR_E = np.array(_PAIR_E)
_PAIR_LEN = (_RANK_FULL[_PAIR_E] - _RANK_FULL[_PAIR_S]).astype(np.float32)
_ADJ = np.array([i for i in range(1, len(_PAIR_B))
                 if _PAIR_B[i] == _PAIR_B[i - 1]])
_VIOL_MAT = np.zeros((_MAX_BINS + 1, len(_ADJ)), np.float32)
for _i, _a in enumerate(_ADJ):
    _VIOL_MAT[_PAIR_B[_a], _i] = 1.0
_ECE_MAT = np.zeros((_MAX_BINS + 1, len(_PAIR_B)), np.float32)
for _i, _b in enumerate(_PAIR_B):
    _ECE_MAT[_b, _i] = 1.0


def _stage_a_body(logits_ref, labels_ref, packed_ref):
    x = logits_ref[0]
    m = jnp.max(x, axis=1, keepdims=True)
    q = jnp.exp(x - m)
    ones = jnp.ones((_C, 1), jnp.float32)
    s = jax.lax.dot_general(q, ones, (((1,), (0,)), ((), ())),
                            preferred_element_type=jnp.float32)
    conf = 1.0 / s[:, 0]
    lab = labels_ref[0, 0]
    lane = lax.broadcasted_iota(jnp.int32, x.shape, 1)
    xl = jnp.max(jnp.where(lane == lab[:, None], x, -jnp.inf), axis=1)
    corr = xl == m[:, 0]
    packed_ref[0, 0] = jnp.where(corr, -conf, conf)


def _stage_a(logits, labels):
    logits3 = logits.reshape(_GRID, _ROWS, _C)
    labels3 = labels.reshape(_GRID, 1, _ROWS)
    packed = pl.pallas_call(
        _stage_a_body,
        grid=(_GRID,),
        in_specs=[
            pl.BlockSpec((1, _ROWS, _C), lambda i: (i, 0, 0)),
            pl.BlockSpec((1, 1, _ROWS), lambda i: (i, 0, 0)),
        ],
        out_specs=pl.BlockSpec((1, 1, _ROWS), lambda i: (i, 0, 0)),
        out_shape=jax.ShapeDtypeStruct((_GRID, 1, _ROWS), jnp.float32),
    )(logits3, labels3)
    return packed.reshape(_N)


_MESH = plsc.VectorSubcoreMesh(core_axis_name="c", subcore_axis_name="s")


def _wid():
    return lax.axis_index("s") * 2 + lax.axis_index("c")


def _search_le(list_ref, v):
    lo = jnp.zeros((16,), jnp.int32)
    step = _LP // 2
    while step >= 1:
        probe = lo + (step - 1)
        t = plsc.load_gather(list_ref, [probe])
        lo = lo + jnp.where(t <= v, step, 0)
        step //= 2
    return lo


def _search_le4(list_ref, vs):
    los = [jnp.zeros((16,), jnp.int32) for _ in vs]
    step = _LP // 2
    while step >= 1:
        ts = [plsc.load_gather(list_ref, [lo + (step - 1)]) for lo in los]
        los = [lo + jnp.where(t <= v, step, 0)
               for lo, t, v in zip(los, ts, vs)]
        step //= 2
    return los


def _zero_ref(ref, nwords):
    z = jnp.zeros((16,), ref.dtype)

    def body(i, _):
        ref[pl.ds(i * 16, 16)] = z
        return 0

    lax.fori_loop(0, nwords // 16, body, 0)


@functools.partial(
    pl.kernel,
    out_type=jax.ShapeDtypeStruct((_NW, _HB), jnp.int32),
    mesh=_MESH,
    compiler_params=pltpu.CompilerParams(needs_layout_passes=False),
    scratch_types=[
        pltpu.VMEM((_CHUNK,), jnp.int32),
        pltpu.VMEM((_HB,), jnp.int32),
    ],
)
def _pass_a(key_hbm, out_hbm, keys_v, hist_v):
    w = _wid()
    pltpu.sync_copy(key_hbm.at[pl.ds(w * _CHUNK, _CHUNK)], keys_v)
    _zero_ref(hist_v, _HB)
    one = jnp.ones((16,), jnp.int32)
    full = jnp.ones((16,), jnp.bool_)

    def body(i, _):
        for j in range(4):
            k = jnp.bitwise_and(keys_v[pl.ds((i * 4 + j) * 16, 16)],
                                jnp.int32(0x7FFFFFFF))
            b = lax.shift_right_logical(k, 16)
            plsc.addupdate_scatter(hist_v, [b], one, mask=full)
        return 0

    lax.fori_loop(0, _VECS // 4, body, 0)
    pltpu.sync_copy(hist_v, out_hbm.at[w])


def _make_refine(shift):
    @functools.partial(
        pl.kernel,
        out_type=jax.ShapeDtypeStruct((_NW, _KROWS * 256), jnp.int32),
        mesh=_MESH,
        compiler_params=pltpu.CompilerParams(needs_layout_passes=False),
        scratch_types=[
            pltpu.VMEM((_CHUNK,), jnp.int32),
            pltpu.VMEM((_LP,), jnp.int32),
            pltpu.VMEM((_KROWS * 256,), jnp.int32),
        ],
    )
    def refine(key_hbm, list_hbm, out_hbm, keys_v, list_v, hist_v):
        w = _wid()
        pltpu.sync_copy(key_hbm.at[pl.ds(w * _CHUNK, _CHUNK)], keys_v)
        pltpu.sync_copy(list_hbm, list_v)
        _zero_ref(hist_v, _KROWS * 256)
        one = jnp.ones((16,), jnp.int32)

        def body(i, _):
            ks = [jnp.bitwise_and(keys_v[pl.ds((i * 4 + j) * 16, 16)],
                                  jnp.int32(0x7FFFFFFF)) for j in range(4)]
            pfxs = [lax.shift_right_logical(k, 8 + shift) for k in ks]
            ss = _search_le4(list_v, pfxs)
            for k, pfx, s in zip(ks, pfxs, ss):
                slot = jnp.maximum(s - 1, 0)
                pv = plsc.load_gather(list_v, [slot])
                valid = jnp.logical_and(s >= 1, pv == pfx)
                dig = jnp.bitwise_and(lax.shift_right_logical(k, shift), 255)
                flat = jnp.where(valid, slot * 256 + dig, 0)
                plsc.addupdate_scatter(hist_v, [flat], one, mask=valid)
            return 0

        lax.fori_loop(0, _VECS // 4, body, 0)
        pltpu.sync_copy(hist_v, out_hbm.at[w])

    return refine


_pass_b = _make_refine(8)
_pass_c = _make_refine(0)


@functools.partial(
    pl.kernel,
    out_type=jax.ShapeDtypeStruct((_NW, 5 * _LP), jnp.float32),
    mesh=_MESH,
    compiler_params=pltpu.CompilerParams(needs_layout_passes=False),
    scratch_types=[
        pltpu.VMEM((_CHUNK,), jnp.int32),
        pltpu.VMEM((_LP,), jnp.int32),
        pltpu.VMEM((5 * _LP,), jnp.float32),
    ],
)
def _pass_d(key_hbm, list_hbm, out_hbm, keys_v, list_v, seg_v):
    w = _wid()
    base = w * _CHUNK
    pltpu.sync_copy(key_hbm.at[pl.ds(base, _CHUNK)], keys_v)
    pltpu.sync_copy(list_hbm, list_v)
    _zero_ref(seg_v, 5 * _LP)
    onef = jnp.ones((16,), jnp.float32)
    lanes = lax.iota(jnp.int32, 16)

    def body(i, _):
        kraw = [keys_v[pl.ds((i * 4 + j) * 16, 16)] for j in range(4)]
        ks = [jnp.bitwise_and(k, jnp.int32(0x7FFFFFFF)) for k in kraw]
        ss = _search_le4(list_v, ks)
        for j, (kr, k, s) in enumerate(zip(kraw, ks, ss)):
            cf = plsc.bitcast(k, jnp.float32)
            cr = jnp.where(kr < 0, 1.0, 0.0).astype(jnp.float32)
            idx = base + (i * 4 + j) * 16 + lanes
            inb = idx < _N
            s = jnp.minimum(s, _KP)
            plsc.addupdate_scatter(seg_v, [s], onef, mask=inb)
            plsc.addupdate_scatter(seg_v, [_LP + s], cf, mask=inb)
            plsc.addupdate_scatter(seg_v, [2 * _LP + s], cr, mask=inb)
            slot = jnp.maximum(s - 1, 0)
            tv = plsc.load_gather(list_v, [slot])
            tie = jnp.logical_and(jnp.logical_and(s >= 1, tv == k), inb)
            plsc.addupdate_scatter(seg_v, [3 * _LP + slot], onef, mask=tie)
            plsc.addupdate_scatter(seg_v, [4 * _LP + slot], cr, mask=tie)
        return 0

    lax.fori_loop(0, _VECS // 4, body, 0)
    pltpu.sync_copy(seg_v, out_hbm.at[w])


def kernel(logits, labels):
    packed = _stage_a(logits, labels)
    ks = lax.bitcast_convert_type(packed, jnp.int32)

    padlen = _PAD_N - _N
    key_p = jnp.concatenate([ks, jnp.full((padlen,), _SENT, jnp.int32)])

    ranks = jnp.asarray(_RANKS)

    hist_a = _pass_a(key_p).sum(axis=0)
    cum_a = jnp.cumsum(hist_a)
    p16 = jnp.searchsorted(cum_a, ranks, side="right").astype(jnp.int32)
    base16 = jnp.where(p16 > 0, cum_a[jnp.maximum(p16 - 1, 0)], 0)

    def padlist(v):
        return jnp.concatenate(
            [v, jnp.full((_LP - _KP,), _SENT, jnp.int32)])

    hist_b = _pass_b(key_p, padlist(p16)).sum(axis=0).reshape(_KROWS, 256)
    cum_b = jnp.cumsum(hist_b[:_KP], axis=1)
    rs_b = jnp.searchsorted(p16, p16, side="right").astype(jnp.int32) - 1
    row_b = cum_b[rs_b]
    m_b = (ranks - base16)[:, None]
    d_b = (row_b <= m_b).sum(axis=1).astype(jnp.int32)
    base24 = base16 + jnp.where(
        d_b > 0, row_b[jnp.arange(_KP), jnp.maximum(d_b - 1, 0)], 0)
    p24 = p16 * 256 + d_b

    hist_c = _pass_c(key_p, padlist(p24)).sum(axis=0).reshape(_KROWS, 256)
    cum_c = jnp.cumsum(hist_c[:_KP], axis=1)
    rs_c = jnp.searchsorted(p24, p24, side="right").astype(jnp.int32) - 1
    row_c = cum_c[rs_c]
    m_c = (ranks - base24)[:, None]
    d_c = (row_c <= m_c).sum(axis=1).astype(jnp.int32)
    tkeys = p24 * 256 + d_c

    segs = _pass_d(key_p, padlist(tkeys)).sum(axis=0).reshape(5, _LP)
    seg_cnt, seg_conf, seg_corr = segs[0], segs[1], segs[2]
    eq_cnt, eq_corr = segs[3], segs[4]

    cum_cnt = jnp.cumsum(seg_cnt)
    cum_conf = jnp.cumsum(seg_conf)
    cum_corr = jnp.cumsum(seg_corr)

    rs_t = jnp.searchsorted(tkeys, tkeys, side="right").astype(jnp.int32) - 1
    take = ranks.astype(jnp.float32) - cum_cnt[:_KP]
    conf_val = lax.bitcast_convert_type(tkeys, jnp.float32)
    ecnt = eq_cnt[rs_t]
    tie_avg = jnp.where(ecnt > 0, eq_corr[rs_t] / jnp.maximum(ecnt, 1.0), 0.0)
    pc_in = cum_conf[:_KP] + take * conf_val
    pr_in = cum_corr[:_KP] + take * tie_avg

    total_conf = cum_conf[_KP]
    total_corr = cum_corr[_KP]
    pc = jnp.concatenate([jnp.zeros((1,)), pc_in, total_conf[None]])
    pr = jnp.concatenate([jnp.zeros((1,)), pr_in, total_corr[None]])

    pair_s = jnp.asarray(_PAIR_S)
    pair_e = jnp.asarray(_PAIR_E)
    ln = jnp.asarray(_PAIR_LEN)
    avg_a = (pr[pair_e] - pr[pair_s]) / ln
    avg_c = (pc[pair_e] - pc[pair_s]) / ln

    adj = jnp.asarray(_ADJ)
    viol = (avg_a[adj] < avg_a[adj - 1]).astype(jnp.float32)
    viol_b = jnp.asarray(_VIOL_MAT) @ viol
    b_arr = jnp.arange(_MAX_BINS + 1)
    n_bins = jnp.min(jnp.where(viol_b > 0, b_arr - 1, _MAX_BINS))

    terms = (ln / _N) * jnp.abs(avg_c - avg_a)
    ece_b = jnp.asarray(_ECE_MAT) @ terms
    ece_b = ece_b.at[1].set(0.0)
    return ece_b[n_bins].astype(jnp.float32)


# speedup vs baseline: 1.1820x; 1.1352x over previous
import functools

import jax
import jax.numpy as jnp
import numpy as np
from jax import lax
from jax.experimental import pallas as pl
from jax.experimental.pallas import tpu as pltpu
from jax.experimental.pallas import tpu_sc as plsc

_N = 100000
_C = 128
_MAX_BINS = 32
_ROWS = 1000
_GRID = _N // _ROWS

_NW = 32
_CHUNK = 3136
_VECS = _CHUNK // 16
_PAD_N = _NW * _CHUNK
_SENT = np.int32(2**31 - 1)

_LP = 512
_HB = 1024
_HOFF = 0x3BF0


def _bin_edges(n, n_bins):
    spb = n // n_bins
    rem = n % n_bins
    edges = []
    start = 0
    for i in range(n_bins):
        size = spb + (1 if i < rem else 0)
        end = start + size
        if start >= n:
            break
        edges.append((start, end))
        start = end
    return edges


_RANKS = np.array(
    sorted({s for b in range(1, _MAX_BINS + 1)
            for (s, e) in _bin_edges(_N, b)} - {0}), dtype=np.int32)
_KP = len(_RANKS)
_KROWS = 400

_RANK_FULL = np.concatenate([[0], _RANKS, [_N]])
_RANK_POS = {int(r): i for i, r in enumerate(_RANK_FULL)}

_PAIR_B, _PAIR_S, _PAIR_E = [], [], []
for _b in range(1, _MAX_BINS + 1):
    for _s, _e in _bin_edges(_N, _b):
        _PAIR_B.append(_b)
        _PAIR_S.append(_RANK_POS[_s])
        _PAIR_E.append(_RANK_POS[_e])
_PAIR_B = np.array(_PAIR_B)
_PAIR_S = np.array(_PAIR_S)
_PAIR_E = np.array(_PAIR_E)
_PAIR_LEN = (_RANK_FULL[_PAIR_E] - _RANK_FULL[_PAIR_S]).astype(np.float32)
_ADJ = np.array([i for i in range(1, len(_PAIR_B))
                 if _PAIR_B[i] == _PAIR_B[i - 1]])
_VIOL_MAT = np.zeros((_MAX_BINS + 1, len(_ADJ)), np.float32)
for _i, _a in enumerate(_ADJ):
    _VIOL_MAT[_PAIR_B[_a], _i] = 1.0
_ECE_MAT = np.zeros((_MAX_BINS + 1, len(_PAIR_B)), np.float32)
for _i, _b in enumerate(_PAIR_B):
    _ECE_MAT[_b, _i] = 1.0


def _stage_a_body(logits_ref, labels_ref, packed_ref):
    x = logits_ref[0]
    m = jnp.max(x, axis=1, keepdims=True)
    q = jnp.exp(x - m)
    ones = jnp.ones((_C, 1), jnp.float32)
    s = jax.lax.dot_general(q, ones, (((1,), (0,)), ((), ())),
                            preferred_element_type=jnp.float32)
    conf = 1.0 / s[:, 0]
    lab = labels_ref[0, 0]
    lane = lax.broadcasted_iota(jnp.int32, x.shape, 1)
    xl = jnp.max(jnp.where(lane == lab[:, None], x, -jnp.inf), axis=1)
    corr = xl == m[:, 0]
    packed_ref[0, 0] = jnp.where(corr, -conf, conf)


def _stage_a(logits, labels):
    logits3 = logits.reshape(_GRID, _ROWS, _C)
    labels3 = labels.reshape(_GRID, 1, _ROWS)
    packed = pl.pallas_call(
        _stage_a_body,
        grid=(_GRID,),
        in_specs=[
            pl.BlockSpec((1, _ROWS, _C), lambda i: (i, 0, 0)),
            pl.BlockSpec((1, 1, _ROWS), lambda i: (i, 0, 0)),
        ],
        out_specs=pl.BlockSpec((1, 1, _ROWS), lambda i: (i, 0, 0)),
        out_shape=jax.ShapeDtypeStruct((_GRID, 1, _ROWS), jnp.float32),
    )(logits3, labels3)
    return packed.reshape(_N)


_MESH = plsc.VectorSubcoreMesh(core_axis_name="c", subcore_axis_name="s")


def _wid():
    return lax.axis_index("s") * 2 + lax.axis_index("c")


def _search_le(list_ref, v):
    lo = jnp.zeros((16,), jnp.int32)
    step = _LP // 2
    while step >= 1:
        probe = lo + (step - 1)
        t = plsc.load_gather(list_ref, [probe])
        lo = lo + jnp.where(t <= v, step, 0)
        step //= 2
    return lo


def _search_le4(list_ref, vs):
    los = [jnp.zeros((16,), jnp.int32) for _ in vs]
    step = _LP // 2
    while step >= 1:
        ts = [plsc.load_gather(list_ref, [lo + (step - 1)]) for lo in los]
        los = [lo + jnp.where(t <= v, step, 0)
               for lo, t, v in zip(los, ts, vs)]
        step //= 2
    return los


def _zero_ref(ref, nwords):
    z = jnp.zeros((16,), ref.dtype)

    def body(i, _):
        ref[pl.ds(i * 16, 16)] = z
        return 0

    lax.fori_loop(0, nwords // 16, body, 0)


@functools.partial(
    pl.kernel,
    out_type=jax.ShapeDtypeStruct((_NW, _HB), jnp.int32),
    mesh=_MESH,
    compiler_params=pltpu.CompilerParams(needs_layout_passes=False),
    scratch_types=[
        pltpu.VMEM((_CHUNK,), jnp.int32),
        pltpu.VMEM((_HB,), jnp.int32),
    ],
)
def _pass_a(key_hbm, out_hbm, keys_v, hist_v):
    w = _wid()
    pltpu.sync_copy(key_hbm.at[pl.ds(w * _CHUNK, _CHUNK)], keys_v)
    _zero_ref(hist_v, _HB)
    one = jnp.ones((16,), jnp.int32)
    full = jnp.ones((16,), jnp.bool_)

    def body(i, _):
        for j in range(4):
            k = jnp.bitwise_and(keys_v[pl.ds((i * 4 + j) * 16, 16)],
                                jnp.int32(0x7FFFFFFF))
            b = lax.shift_right_logical(k, 16) - _HOFF
            b = jnp.clip(b, 0, _HB - 1)
            plsc.addupdate_scatter(hist_v, [b], one, mask=full)
        return 0

    lax.fori_loop(0, _VECS // 4, body, 0)
    pltpu.sync_copy(hist_v, out_hbm.at[w])


def _make_refine(shift):
    @functools.partial(
        pl.kernel,
        out_type=jax.ShapeDtypeStruct((_NW, _KROWS * 256), jnp.int32),
        mesh=_MESH,
        compiler_params=pltpu.CompilerParams(needs_layout_passes=False),
        scratch_types=[
            pltpu.VMEM((_CHUNK,), jnp.int32),
            pltpu.VMEM((_LP,), jnp.int32),
            pltpu.VMEM((_KROWS * 256,), jnp.int32),
        ],
    )
    def refine(key_hbm, list_hbm, out_hbm, keys_v, list_v, hist_v):
        w = _wid()
        pltpu.sync_copy(key_hbm.at[pl.ds(w * _CHUNK, _CHUNK)], keys_v)
        pltpu.sync_copy(list_hbm, list_v)
        _zero_ref(hist_v, _KROWS * 256)
        one = jnp.ones((16,), jnp.int32)

        def body(i, _):
            ks = [jnp.bitwise_and(keys_v[pl.ds((i * 4 + j) * 16, 16)],
                                  jnp.int32(0x7FFFFFFF)) for j in range(4)]
            pfxs = [lax.shift_right_logical(k, 8 + shift) for k in ks]
            ss = _search_le4(list_v, pfxs)
            for k, pfx, s in zip(ks, pfxs, ss):
                slot = jnp.maximum(s - 1, 0)
                pv = plsc.load_gather(list_v, [slot])
                valid = jnp.logical_and(s >= 1, pv == pfx)
                dig = jnp.bitwise_and(lax.shift_right_logical(k, shift), 255)
                flat = jnp.where(valid, slot * 256 + dig, 0)
                plsc.addupdate_scatter(hist_v, [flat], one, mask=valid)
            return 0

        lax.fori_loop(0, _VECS // 4, body, 0)
        pltpu.sync_copy(hist_v, out_hbm.at[w])

    return refine


_pass_b = _make_refine(8)
_pass_c = _make_refine(0)


@functools.partial(
    pl.kernel,
    out_type=jax.ShapeDtypeStruct((_NW, 5 * _LP), jnp.float32),
    mesh=_MESH,
    compiler_params=pltpu.CompilerParams(needs_layout_passes=False),
    scratch_types=[
        pltpu.VMEM((_CHUNK,), jnp.int32),
        pltpu.VMEM((_LP,), jnp.int32),
        pltpu.VMEM((5 * _LP,), jnp.float32),
    ],
)
def _pass_d(key_hbm, list_hbm, out_hbm, keys_v, list_v, seg_v):
    w = _wid()
    base = w * _CHUNK
    pltpu.sync_copy(key_hbm.at[pl.ds(base, _CHUNK)], keys_v)
    pltpu.sync_copy(list_hbm, list_v)
    _zero_ref(seg_v, 5 * _LP)
    onef = jnp.ones((16,), jnp.float32)
    lanes = lax.iota(jnp.int32, 16)

    def body(i, _):
        kraw = [keys_v[pl.ds((i * 4 + j) * 16, 16)] for j in range(4)]
        ks = [jnp.bitwise_and(k, jnp.int32(0x7FFFFFFF)) for k in kraw]
        ss = _search_le4(list_v, ks)
        for j, (kr, k, s) in enumerate(zip(kraw, ks, ss)):
            cf = plsc.bitcast(k, jnp.float32)
            cr = jnp.where(kr < 0, 1.0, 0.0).astype(jnp.float32)
            idx = base + (i * 4 + j) * 16 + lanes
            inb = idx < _N
            s = jnp.minimum(s, _KP)
            plsc.addupdate_scatter(seg_v, [s], onef, mask=inb)
            plsc.addupdate_scatter(seg_v, [_LP + s], cf, mask=inb)
            plsc.addupdate_scatter(seg_v, [2 * _LP + s], cr, mask=inb)
            slot = jnp.maximum(s - 1, 0)
            tv = plsc.load_gather(list_v, [slot])
            tie = jnp.logical_and(jnp.logical_and(s >= 1, tv == k), inb)
            plsc.addupdate_scatter(seg_v, [3 * _LP + slot], onef, mask=tie)
            plsc.addupdate_scatter(seg_v, [4 * _LP + slot], cr, mask=tie)
        return 0

    lax.fori_loop(0, _VECS // 4, body, 0)
    pltpu.sync_copy(seg_v, out_hbm.at[w])


def kernel(logits, labels):
    packed = _stage_a(logits, labels)
    ks = lax.bitcast_convert_type(packed, jnp.int32)

    padlen = _PAD_N - _N
    key_p = jnp.concatenate([ks, jnp.full((padlen,), _SENT, jnp.int32)])

    ranks = jnp.asarray(_RANKS)

    hist_a = _pass_a(key_p).sum(axis=0)
    cum_a = jnp.cumsum(hist_a)
    ib = jnp.searchsorted(cum_a, ranks, side="right").astype(jnp.int32)
    base16 = jnp.where(ib > 0, cum_a[jnp.maximum(ib - 1, 0)], 0)
    p16 = ib + _HOFF

    def padlist(v):
        return jnp.concatenate(
            [v, jnp.full((_LP - _KP,), _SENT, jnp.int32)])

    hist_b = _pass_b(key_p, padlist(p16)).sum(axis=0).reshape(_KROWS, 256)
    cum_b = jnp.cumsum(hist_b[:_KP], axis=1)
    rs_b = jnp.searchsorted(p16, p16, side="right").astype(jnp.int32) - 1
    row_b = cum_b[rs_b]
    m_b = (ranks - base16)[:, None]
    d_b = (row_b <= m_b).sum(axis=1).astype(jnp.int32)
    base24 = base16 + jnp.where(
        d_b > 0, row_b[jnp.arange(_KP), jnp.maximum(d_b - 1, 0)], 0)
    p24 = p16 * 256 + d_b

    hist_c = _pass_c(key_p, padlist(p24)).sum(axis=0).reshape(_KROWS, 256)
    cum_c = jnp.cumsum(hist_c[:_KP], axis=1)
    rs_c = jnp.searchsorted(p24, p24, side="right").astype(jnp.int32) - 1
    row_c = cum_c[rs_c]
    m_c = (ranks - base24)[:, None]
    d_c = (row_c <= m_c).sum(axis=1).astype(jnp.int32)
    tkeys = p24 * 256 + d_c

    segs = _pass_d(key_p, padlist(tkeys)).sum(axis=0).reshape(5, _LP)
    seg_cnt, seg_conf, seg_corr = segs[0], segs[1], segs[2]
    eq_cnt, eq_corr = segs[3], segs[4]

    cum_cnt = jnp.cumsum(seg_cnt)
    cum_conf = jnp.cumsum(seg_conf)
    cum_corr = jnp.cumsum(seg_corr)

    rs_t = jnp.searchsorted(tkeys, tkeys, side="right").astype(jnp.int32) - 1
    take = ranks.astype(jnp.float32) - cum_cnt[:_KP]
    conf_val = lax.bitcast_convert_type(tkeys, jnp.float32)
    ecnt = eq_cnt[rs_t]
    tie_avg = jnp.where(ecnt > 0, eq_corr[rs_t] / jnp.maximum(ecnt, 1.0), 0.0)
    pc_in = cum_conf[:_KP] + take * conf_val
    pr_in = cum_corr[:_KP] + take * tie_avg

    total_conf = cum_conf[_KP]
    total_corr = cum_corr[_KP]
    pc = jnp.concatenate([jnp.zeros((1,)), pc_in, total_conf[None]])
    pr = jnp.concatenate([jnp.zeros((1,)), pr_in, total_corr[None]])

    pair_s = jnp.asarray(_PAIR_S)
    pair_e = jnp.asarray(_PAIR_E)
    ln = jnp.asarray(_PAIR_LEN)
    avg_a = (pr[pair_e] - pr[pair_s]) / ln
    avg_c = (pc[pair_e] - pc[pair_s]) / ln

    adj = jnp.asarray(_ADJ)
    viol = (avg_a[adj] < avg_a[adj - 1]).astype(jnp.float32)
    viol_b = jnp.asarray(_VIOL_MAT) @ viol
    b_arr = jnp.arange(_MAX_BINS + 1)
    n_bins = jnp.min(jnp.where(viol_b > 0, b_arr - 1, _MAX_BINS))

    terms = (ln / _N) * jnp.abs(avg_c - avg_a)
    ece_b = jnp.asarray(_ECE_MAT) @ terms
    ece_b = ece_b.at[1].set(0.0)
    return ece_b[n_bins].astype(jnp.float32)


# speedup vs baseline: 1.2043x; 1.0189x over previous
import functools

import jax
import jax.numpy as jnp
import numpy as np
from jax import lax
from jax.experimental import pallas as pl
from jax.experimental.pallas import tpu as pltpu
from jax.experimental.pallas import tpu_sc as plsc

_N = 100000
_C = 128
_MAX_BINS = 32
_ROWS = 1000
_GRID = _N // _ROWS

_NW = 32
_CHUNK = 3136
_VECS = _CHUNK // 16
_PAD_N = _NW * _CHUNK
_SENT = np.int32(2**31 - 1)

_LP = 512
_HB = 1024
_HOFF = 0x3BF0


def _bin_edges(n, n_bins):
    spb = n // n_bins
    rem = n % n_bins
    edges = []
    start = 0
    for i in range(n_bins):
        size = spb + (1 if i < rem else 0)
        end = start + size
        if start >= n:
            break
        edges.append((start, end))
        start = end
    return edges


_RANKS = np.array(
    sorted({s for b in range(1, _MAX_BINS + 1)
            for (s, e) in _bin_edges(_N, b)} - {0}), dtype=np.int32)
_KP = len(_RANKS)
_KROWS = 400

_RANK_FULL = np.concatenate([[0], _RANKS, [_N]])
_RANK_POS = {int(r): i for i, r in enumerate(_RANK_FULL)}

_PAIR_B, _PAIR_S, _PAIR_E = [], [], []
for _b in range(1, _MAX_BINS + 1):
    for _s, _e in _bin_edges(_N, _b):
        _PAIR_B.append(_b)
        _PAIR_S.append(_RANK_POS[_s])
        _PAIR_E.append(_RANK_POS[_e])
_PAIR_B = np.array(_PAIR_B)
_PAIR_S = np.array(_PAIR_S)
_PAIR_E = np.array(_PAIR_E)
_PAIR_LEN = (_RANK_FULL[_PAIR_E] - _RANK_FULL[_PAIR_S]).astype(np.float32)
_ADJ = np.array([i for i in range(1, len(_PAIR_B))
                 if _PAIR_B[i] == _PAIR_B[i - 1]])
_VIOL_MAT = np.zeros((_MAX_BINS + 1, len(_ADJ)), np.float32)
for _i, _a in enumerate(_ADJ):
    _VIOL_MAT[_PAIR_B[_a], _i] = 1.0
_ECE_MAT = np.zeros((_MAX_BINS + 1, len(_PAIR_B)), np.float32)
for _i, _b in enumerate(_PAIR_B):
    _ECE_MAT[_b, _i] = 1.0


def _stage_a_body(logits_ref, labels_ref, packed_ref):
    x = logits_ref[0]
    m = jnp.max(x, axis=1, keepdims=True)
    q = jnp.exp(x - m)
    ones = jnp.ones((_C, 1), jnp.float32)
    s = jax.lax.dot_general(q, ones, (((1,), (0,)), ((), ())),
                            preferred_element_type=jnp.float32)
    conf = 1.0 / s[:, 0]
    lab = labels_ref[0, 0]
    lane = lax.broadcasted_iota(jnp.int32, x.shape, 1)
    xl = jnp.max(jnp.where(lane == lab[:, None], x, -jnp.inf), axis=1)
    corr = xl == m[:, 0]
    packed_ref[0, 0] = jnp.where(corr, -conf, conf)


def _stage_a(logits, labels):
    logits3 = logits.reshape(_GRID, _ROWS, _C)
    labels3 = labels.reshape(_GRID, 1, _ROWS)
    packed = pl.pallas_call(
        _stage_a_body,
        grid=(_GRID,),
        in_specs=[
            pl.BlockSpec((1, _ROWS, _C), lambda i: (i, 0, 0)),
            pl.BlockSpec((1, 1, _ROWS), lambda i: (i, 0, 0)),
        ],
        out_specs=pl.BlockSpec((1, 1, _ROWS), lambda i: (i, 0, 0)),
        out_shape=jax.ShapeDtypeStruct((_GRID, 1, _ROWS), jnp.float32),
    )(logits3, labels3)
    return packed.reshape(_N)


_MESH = plsc.VectorSubcoreMesh(core_axis_name="c", subcore_axis_name="s")


def _wid():
    return lax.axis_index("s") * 2 + lax.axis_index("c")


def _search_le(list_ref, v):
    lo = jnp.zeros((16,), jnp.int32)
    step = _LP // 2
    while step >= 1:
        probe = lo + (step - 1)
        t = plsc.load_gather(list_ref, [probe])
        lo = lo + jnp.where(t <= v, step, 0)
        step //= 2
    return lo


def _search_le4(list_ref, vs):
    los = [jnp.zeros((16,), jnp.int32) for _ in vs]
    step = _LP // 2
    while step >= 1:
        ts = [plsc.load_gather(list_ref, [lo + (step - 1)]) for lo in los]
        los = [lo + jnp.where(t <= v, step, 0)
               for lo, t, v in zip(los, ts, vs)]
        step //= 2
    return los


def _zero_ref2d(ref, rows):
    z = jnp.zeros((16,), ref.dtype)

    def body(i, _):
        ref[i // 8, pl.ds((i % 8) * 16, 16)] = z
        return 0

    lax.fori_loop(0, rows * 8, body, 0)


def _zero_ref(ref, nwords):
    z = jnp.zeros((16,), ref.dtype)

    def body(i, _):
        ref[pl.ds(i * 16, 16)] = z
        return 0

    lax.fori_loop(0, nwords // 16, body, 0)


@functools.partial(
    pl.kernel,
    out_type=jax.ShapeDtypeStruct((_NW, _HB), jnp.int32),
    mesh=_MESH,
    compiler_params=pltpu.CompilerParams(needs_layout_passes=False),
    scratch_types=[
        pltpu.VMEM((_CHUNK,), jnp.int32),
        pltpu.VMEM((_HB,), jnp.int32),
    ],
)
def _pass_a(key_hbm, out_hbm, keys_v, hist_v):
    w = _wid()
    pltpu.sync_copy(key_hbm.at[pl.ds(w * _CHUNK, _CHUNK)], keys_v)
    _zero_ref(hist_v, _HB)
    one = jnp.ones((16,), jnp.int32)
    full = jnp.ones((16,), jnp.bool_)

    def body(i, _):
        for j in range(4):
            k = jnp.bitwise_and(keys_v[pl.ds((i * 4 + j) * 16, 16)],
                                jnp.int32(0x7FFFFFFF))
            b = lax.shift_right_logical(k, 16) - _HOFF
            b = jnp.clip(b, 0, _HB - 1)
            plsc.addupdate_scatter(hist_v, [b], one, mask=full)
        return 0

    lax.fori_loop(0, _VECS // 4, body, 0)
    pltpu.sync_copy(hist_v, out_hbm.at[w])


_HROWS = _KROWS * 2
_HPAD = 896


def _make_refine(shift):
    @functools.partial(
        pl.kernel,
        out_type=jax.ShapeDtypeStruct((2, _HPAD, 128), jnp.int32),
        mesh=_MESH,
        compiler_params=pltpu.CompilerParams(needs_layout_passes=False),
        scratch_types=[
            pltpu.VMEM((_CHUNK,), jnp.int32),
            pltpu.VMEM((_LP,), jnp.int32),
            pltpu.VMEM((_HPAD, 128), jnp.int32),
            pltpu.VMEM((7, 128), jnp.int32),
            pltpu.VMEM_SHARED((_HPAD, 128), jnp.int32),
        ],
    )
    def refine(key_hbm, list_hbm, out_hbm, keys_v, list_v, hist_v,
               idx2_v, shared_v):
        w = _wid()
        sid = lax.axis_index("s")
        core = lax.axis_index("c")
        pltpu.sync_copy(key_hbm.at[pl.ds(w * _CHUNK, _CHUNK)], keys_v)
        pltpu.sync_copy(list_hbm, list_v)
        _zero_ref2d(hist_v, _HPAD)
        lanes = lax.iota(jnp.int32, 16)

        def idxbody(i, _):
            r = i // 8
            c = (i % 8) * 16
            idx2_v[r, pl.ds(c, 16)] = r * 128 + c + lanes
            return 0

        lax.fori_loop(0, 56, idxbody, 0)

        stripe = _HPAD // 16
        pltpu.sync_copy(hist_v.at[pl.ds(0, stripe)],
                        shared_v.at[pl.ds(sid * stripe, stripe)])
        plsc.subcore_barrier()

        one = jnp.ones((16,), jnp.int32)

        def body(i, _):
            ks = [jnp.bitwise_and(keys_v[pl.ds((i * 4 + j) * 16, 16)],
                                  jnp.int32(0x7FFFFFFF)) for j in range(4)]
            pfxs = [lax.shift_right_logical(k, 8 + shift) for k in ks]
            ss = _search_le4(list_v, pfxs)
            for k, pfx, s in zip(ks, pfxs, ss):
                slot = jnp.maximum(s - 1, 0)
                pv = plsc.load_gather(list_v, [slot])
                valid = jnp.logical_and(s >= 1, pv == pfx)
                dig = jnp.bitwise_and(lax.shift_right_logical(k, shift), 255)
                row = jnp.where(valid, slot * 2
                                + lax.shift_right_logical(dig, 7), 0)
                col = jnp.bitwise_and(dig, 127)
                plsc.addupdate_scatter(hist_v, [row, col], one, mask=valid)
            return 0

        lax.fori_loop(0, _VECS // 4, body, 0)

        for j in range(7):
            pltpu.sync_copy(hist_v.at[pl.ds(j * 128, 128)],
                            shared_v.at[idx2_v.at[j]], add=True)
        plsc.subcore_barrier()
        pltpu.sync_copy(shared_v.at[pl.ds(sid * stripe, stripe)],
                        out_hbm.at[core, pl.ds(sid * stripe, stripe)])

    return refine


_pass_b = _make_refine(8)
_pass_c = _make_refine(0)


@functools.partial(
    pl.kernel,
    out_type=jax.ShapeDtypeStruct((_NW, 5 * _LP), jnp.float32),
    mesh=_MESH,
    compiler_params=pltpu.CompilerParams(needs_layout_passes=False),
    scratch_types=[
        pltpu.VMEM((_CHUNK,), jnp.int32),
        pltpu.VMEM((_LP,), jnp.int32),
        pltpu.VMEM((5 * _LP,), jnp.float32),
    ],
)
def _pass_d(key_hbm, list_hbm, out_hbm, keys_v, list_v, seg_v):
    w = _wid()
    base = w * _CHUNK
    pltpu.sync_copy(key_hbm.at[pl.ds(base, _CHUNK)], keys_v)
    pltpu.sync_copy(list_hbm, list_v)
    _zero_ref(seg_v, 5 * _LP)
    onef = jnp.ones((16,), jnp.float32)
    lanes = lax.iota(jnp.int32, 16)

    def body(i, _):
        kraw = [keys_v[pl.ds((i * 4 + j) * 16, 16)] for j in range(4)]
        ks = [jnp.bitwise_and(k, jnp.int32(0x7FFFFFFF)) for k in kraw]
        ss = _search_le4(list_v, ks)
        for j, (kr, k, s) in enumerate(zip(kraw, ks, ss)):
            cf = plsc.bitcast(k, jnp.float32)
            cr = jnp.where(kr < 0, 1.0, 0.0).astype(jnp.float32)
            idx = base + (i * 4 + j) * 16 + lanes
            inb = idx < _N
            s = jnp.minimum(s, _KP)
            plsc.addupdate_scatter(seg_v, [s], onef, mask=inb)
            plsc.addupdate_scatter(seg_v, [_LP + s], cf, mask=inb)
            plsc.addupdate_scatter(seg_v, [2 * _LP + s], cr, mask=inb)
            slot = jnp.maximum(s - 1, 0)
            tv = plsc.load_gather(list_v, [slot])
            tie = jnp.logical_and(jnp.logical_and(s >= 1, tv == k), inb)
            plsc.addupdate_scatter(seg_v, [3 * _LP + slot], onef, mask=tie)
            plsc.addupdate_scatter(seg_v, [4 * _LP + slot], cr, mask=tie)
        return 0

    lax.fori_loop(0, _VECS // 4, body, 0)
    pltpu.sync_copy(seg_v, out_hbm.at[w])


def kernel(logits, labels):
    packed = _stage_a(logits, labels)
    ks = lax.bitcast_convert_type(packed, jnp.int32)

    padlen = _PAD_N - _N
    key_p = jnp.concatenate([ks, jnp.full((padlen,), _SENT, jnp.int32)])

    ranks = jnp.asarray(_RANKS)

    hist_a = _pass_a(key_p).sum(axis=0)
    cum_a = jnp.cumsum(hist_a)
    ib = jnp.searchsorted(cum_a, ranks, side="right").astype(jnp.int32)
    base16 = jnp.where(ib > 0, cum_a[jnp.maximum(ib - 1, 0)], 0)
    p16 = ib + _HOFF

    def padlist(v):
        return jnp.concatenate(
            [v, jnp.full((_LP - _KP,), _SENT, jnp.int32)])

    hist_b = _pass_b(key_p, padlist(p16)).sum(axis=0)[:_HROWS].reshape(_KROWS, 256)
    cum_b = jnp.cumsum(hist_b[:_KP], axis=1)
    rs_b = jnp.searchsorted(p16, p16, side="right").astype(jnp.int32) - 1
    row_b = cum_b[rs_b]
    m_b = (ranks - base16)[:, None]
    d_b = (row_b <= m_b).sum(axis=1).astype(jnp.int32)
    base24 = base16 + jnp.where(
        d_b > 0, row_b[jnp.arange(_KP), jnp.maximum(d_b - 1, 0)], 0)
    p24 = p16 * 256 + d_b

    hist_c = _pass_c(key_p, padlist(p24)).sum(axis=0)[:_HROWS].reshape(_KROWS, 256)
    cum_c = jnp.cumsum(hist_c[:_KP], axis=1)
    rs_c = jnp.searchsorted(p24, p24, side="right").astype(jnp.int32) - 1
    row_c = cum_c[rs_c]
    m_c = (ranks - base24)[:, None]
    d_c = (row_c <= m_c).sum(axis=1).astype(jnp.int32)
    tkeys = p24 * 256 + d_c

    segs = _pass_d(key_p, padlist(tkeys)).sum(axis=0).reshape(5, _LP)
    seg_cnt, seg_conf, seg_corr = segs[0], segs[1], segs[2]
    eq_cnt, eq_corr = segs[3], segs[4]

    cum_cnt = jnp.cumsum(seg_cnt)
    cum_conf = jnp.cumsum(seg_conf)
    cum_corr = jnp.cumsum(seg_corr)

    rs_t = jnp.searchsorted(tkeys, tkeys, side="right").astype(jnp.int32) - 1
    take = ranks.astype(jnp.float32) - cum_cnt[:_KP]
    conf_val = lax.bitcast_convert_type(tkeys, jnp.float32)
    ecnt = eq_cnt[rs_t]
    tie_avg = jnp.where(ecnt > 0, eq_corr[rs_t] / jnp.maximum(ecnt, 1.0), 0.0)
    pc_in = cum_conf[:_KP] + take * conf_val
    pr_in = cum_corr[:_KP] + take * tie_avg

    total_conf = cum_conf[_KP]
    total_corr = cum_corr[_KP]
    pc = jnp.concatenate([jnp.zeros((1,)), pc_in, total_conf[None]])
    pr = jnp.concatenate([jnp.zeros((1,)), pr_in, total_corr[None]])

    pair_s = jnp.asarray(_PAIR_S)
    pair_e = jnp.asarray(_PAIR_E)
    ln = jnp.asarray(_PAIR_LEN)
    avg_a = (pr[pair_e] - pr[pair_s]) / ln
    avg_c = (pc[pair_e] - pc[pair_s]) / ln

    adj = jnp.asarray(_ADJ)
    viol = (avg_a[adj] < avg_a[adj - 1]).astype(jnp.float32)
    viol_b = jnp.asarray(_VIOL_MAT) @ viol
    b_arr = jnp.arange(_MAX_BINS + 1)
    n_bins = jnp.min(jnp.where(viol_b > 0, b_arr - 1, _MAX_BINS))

    terms = (ln / _N) * jnp.abs(avg_c - avg_a)
    ece_b = jnp.asarray(_ECE_MAT) @ terms
    ece_b = ece_b.at[1].set(0.0)
    return ece_b[n_bins].astype(jnp.float32)
